# Initial kernel scaffold; baseline (speedup 1.0000x reference)
import jax
import jax.numpy as jnp
from jax.experimental import pallas as pl


def kernel(x, edge_index, batch, sact_cids, W1, b1, Wg1, bg1, Wg2, bg2, Wh, bh):
    raise NotImplementedError("write your pallas kernel here")


# speedup vs baseline: 3.2342x; 3.2342x over previous
import functools

import jax
import jax.numpy as jnp
from jax import lax
from jax.experimental import pallas as pl
from jax.experimental.pallas import tpu as pltpu
from jax.experimental.pallas import tpu_sc as plsc

NC = 2
NS = 16
NCH = 8


@functools.lru_cache(maxsize=None)
def _make_deg_kernel(E: int, NPAD: int):
    EPT = E // (NC * NS)
    K = 40
    NB = EPT // K
    RPT = NPAD // NS
    ZR = 80
    assert EPT % K == 0 and RPT % ZR == 0 and RPT % 8 == 0
    mesh = plsc.VectorSubcoreMesh(core_axis_name="c", subcore_axis_name="s")

    def body(dst_hbm, out_hbm, dst_v, ones_v, zeros_v, acc):
        cid = lax.axis_index("c")
        sid = lax.axis_index("s")
        for r in range(K):
            ones_v[r] = jnp.ones((16,), jnp.float32)
        for r in range(ZR):
            zeros_v[r] = jnp.zeros((16,), jnp.float32)
        for z in range(RPT // ZR):
            pltpu.sync_copy(zeros_v, acc.at[pl.ds(sid * RPT + z * ZR, ZR)])
        plsc.subcore_barrier()
        ebase = cid * (E // NC) + sid * EPT

        def batch(b, carry):
            pltpu.sync_copy(dst_hbm.at[pl.ds(ebase + b * K, K)], dst_v)
            pltpu.sync_copy(ones_v, acc.at[dst_v], add=True)
            return carry

        lax.fori_loop(0, NB, batch, 0)
        plsc.subcore_barrier()
        pltpu.sync_copy(acc.at[pl.ds(sid * RPT, RPT)],
                        out_hbm.at[cid, pl.ds(sid * RPT, RPT)])

    return pl.kernel(
        body,
        out_type=jax.ShapeDtypeStruct((NC, NPAD, 16), jnp.float32),
        mesh=mesh,
        scratch_types=[
            pltpu.VMEM((K,), jnp.int32),
            pltpu.VMEM((K, 16), jnp.float32),
            pltpu.VMEM((ZR, 16), jnp.float32),
            pltpu.VMEM_SHARED((NPAD, 16), jnp.float32),
        ],
    )


@functools.lru_cache(maxsize=None)
def _make_scatter_kernel(E: int, NPAD: int, CW: int):
    EPT = E // NS
    K = 80
    NB = EPT // K
    RPT = NPAD // NS
    ZR = 80
    CPS = NCH // NC
    assert EPT % K == 0 and RPT % ZR == 0 and K % 16 == 0
    mesh = plsc.VectorSubcoreMesh(core_axis_name="c", subcore_axis_name="s")

    def body(g_hbm, src_hbm, dst_hbm, out_hbm,
             src_v, dst_v, gidx_v, rows_v, zeros_v, acc):
        cid = lax.axis_index("c")
        sid = lax.axis_index("s")

        def fill_zero(r, carry):
            for i in range(CW // 16):
                zeros_v[r, pl.ds(i * 16, 16)] = jnp.zeros((16,), jnp.float32)
            return carry

        lax.fori_loop(0, ZR, fill_zero, 0)
        ebase = sid * EPT
        for j in range(CPS):
            chunk = cid * CPS + j
            for z in range(RPT // ZR):
                pltpu.sync_copy(zeros_v, acc.at[pl.ds(sid * RPT + z * ZR, ZR)])
            plsc.subcore_barrier()

            def batch(b, carry):
                off = ebase + b * K
                pltpu.sync_copy(src_hbm.at[pl.ds(off, K)], src_v)
                pltpu.sync_copy(dst_hbm.at[pl.ds(off, K)], dst_v)
                for i in range(K // 16):
                    gidx_v[pl.ds(i * 16, 16)] = (
                        src_v[pl.ds(i * 16, 16)] * NCH + chunk)
                pltpu.sync_copy(g_hbm.at[gidx_v], rows_v)
                pltpu.sync_copy(rows_v, acc.at[dst_v], add=True)
                return carry

            lax.fori_loop(0, NB, batch, 0)
            plsc.subcore_barrier()
            pltpu.sync_copy(acc.at[pl.ds(sid * RPT, RPT)],
                            out_hbm.at[chunk, pl.ds(sid * RPT, RPT)])

    return pl.kernel(
        body,
        out_type=jax.ShapeDtypeStruct((NCH, NPAD, CW), jnp.float32),
        mesh=mesh,
        scratch_types=[
            pltpu.VMEM((K,), jnp.int32),
            pltpu.VMEM((K,), jnp.int32),
            pltpu.VMEM((K,), jnp.int32),
            pltpu.VMEM((K, CW), jnp.float32),
            pltpu.VMEM((ZR, CW), jnp.float32),
            pltpu.VMEM_SHARED((NPAD, CW), jnp.float32),
        ],
    )


_R = 1000


def _dis_from_parts(dp):
    deg = dp[0, :, 0:1] + dp[1, :, 0:1] + 1.0
    return lax.rsqrt(deg)


def _encode_body(x_ref, w1_ref, b1_ref, wg1_ref, degp_ref, out_ref):
    a = jnp.dot(x_ref[...], w1_ref[...], preferred_element_type=jnp.float32)
    a = jnp.maximum(a + b1_ref[...], 0.0)
    t = jnp.dot(a, wg1_ref[...], preferred_element_type=jnp.float32)
    out_ref[...] = _dis_from_parts(degp_ref[...]) * t


def _tc_encode(x, W1, b1, Wg1, degp):
    N, DIN = x.shape
    D = Wg1.shape[0]
    grid = (N // _R,)
    return pl.pallas_call(
        _encode_body,
        grid=grid,
        in_specs=[
            pl.BlockSpec((_R, DIN), lambda i: (i, 0)),
            pl.BlockSpec((DIN, D), lambda i: (0, 0)),
            pl.BlockSpec((1, D), lambda i: (0, 0)),
            pl.BlockSpec((D, D), lambda i: (0, 0)),
            pl.BlockSpec((NC, _R, 16), lambda i: (0, i, 0)),
        ],
        out_specs=pl.BlockSpec((_R, D), lambda i: (i, 0)),
        out_shape=jax.ShapeDtypeStruct((N, D), jnp.float32),
    )(x, W1, b1.reshape(1, D), Wg1, degp)


def _combine(s, g, dis, bg):
    scat = jnp.concatenate([s[c] for c in range(NCH)], axis=1)
    return jnp.maximum(dis * (scat + g) + bg, 0.0)


def _mid_body(s_ref, g_ref, degp_ref, bg_ref, w_ref, out_ref):
    dis = _dis_from_parts(degp_ref[...])
    h = _combine(s_ref[...], g_ref[...], dis, bg_ref[...])
    t = jnp.dot(h, w_ref[...], preferred_element_type=jnp.float32)
    out_ref[...] = dis * t


def _tc_mid(s_cm, g, degp, bg, W):
    N, D = g.shape
    CW = D // NCH
    grid = (N // _R,)
    return pl.pallas_call(
        _mid_body,
        grid=grid,
        in_specs=[
            pl.BlockSpec((NCH, _R, CW), lambda i: (0, i, 0)),
            pl.BlockSpec((_R, D), lambda i: (i, 0)),
            pl.BlockSpec((NC, _R, 16), lambda i: (0, i, 0)),
            pl.BlockSpec((1, D), lambda i: (0, 0)),
            pl.BlockSpec((D, D), lambda i: (0, 0)),
        ],
        out_specs=pl.BlockSpec((_R, D), lambda i: (i, 0)),
        out_shape=jax.ShapeDtypeStruct((N, D), jnp.float32),
    )(s_cm, g, degp, bg.reshape(1, D), W)


def _head_body(s_ref, g_ref, degp_ref, bg_ref, batch_ref, cids_ref, wh_ref,
               bh_ref, loss_ref, logits_ref, pooled_acc, cnt_acc, *, nsteps, nb, nc):
    i = pl.program_id(0)

    @pl.when(i == 0)
    def _init():
        pooled_acc[...] = jnp.zeros_like(pooled_acc)
        cnt_acc[...] = jnp.zeros_like(cnt_acc)

    dis = _dis_from_parts(degp_ref[...])
    h = _combine(s_ref[...], g_ref[...], dis, bg_ref[...])
    bcol = batch_ref[...]
    mask = (bcol == lax.broadcasted_iota(jnp.int32, (_R, nb), 1)
            ).astype(jnp.float32)
    pooled_acc[...] += lax.dot_general(
        mask, h, (((0,), (0,)), ((), ())), preferred_element_type=jnp.float32)
    cnt_acc[...] += jnp.sum(mask, axis=0)[:, None]

    @pl.when(i == nsteps - 1)
    def _fin():
        pooled = pooled_acc[...] / jnp.maximum(cnt_acc[...], 1.0)
        logits = jnp.dot(pooled, wh_ref[...],
                         preferred_element_type=jnp.float32) + bh_ref[...]
        logits_ref[...] = logits
        m = jnp.max(logits, axis=1, keepdims=True)
        lse = jnp.log(jnp.sum(jnp.exp(logits - m), axis=1, keepdims=True)) + m
        logp = logits - lse
        oh = (cids_ref[...] == lax.broadcasted_iota(jnp.int32, (nb, nc), 1)
              ).astype(jnp.float32)
        loss_ref[...] = (-jnp.sum(logp * oh) / nb).reshape(1, 1)


def _tc_head(s_cm, g, degp, bg, batch, cids, Wh, bh):
    N, D = g.shape
    CW = D // NCH
    C = Wh.shape[1]
    B = cids.shape[0]
    nsteps = N // _R
    body = functools.partial(_head_body, nsteps=nsteps, nb=B, nc=C)
    return pl.pallas_call(
        body,
        grid=(nsteps,),
        in_specs=[
            pl.BlockSpec((NCH, _R, CW), lambda i: (0, i, 0)),
            pl.BlockSpec((_R, D), lambda i: (i, 0)),
            pl.BlockSpec((NC, _R, 16), lambda i: (0, i, 0)),
            pl.BlockSpec((1, D), lambda i: (0, 0)),
            pl.BlockSpec((_R, 1), lambda i: (i, 0)),
            pl.BlockSpec((B, 1), lambda i: (0, 0)),
            pl.BlockSpec((D, C), lambda i: (0, 0)),
            pl.BlockSpec((1, C), lambda i: (0, 0)),
        ],
        out_specs=[
            pl.BlockSpec((1, 1), lambda i: (0, 0)),
            pl.BlockSpec((B, C), lambda i: (0, 0)),
        ],
        out_shape=[
            jax.ShapeDtypeStruct((1, 1), jnp.float32),
            jax.ShapeDtypeStruct((B, C), jnp.float32),
        ],
        scratch_shapes=[
            pltpu.VMEM((B, D), jnp.float32),
            pltpu.VMEM((B, 1), jnp.float32),
        ],
    )(s_cm, g, degp, bg.reshape(1, D), batch.reshape(N, 1),
      cids.reshape(B, 1), Wh, bh.reshape(1, C))


def kernel(x, edge_index, batch, sact_cids, W1, b1, Wg1, bg1, Wg2, bg2, Wh, bh):
    N = x.shape[0]
    E = edge_index.shape[1]
    D = Wg1.shape[0]
    CW = D // NCH
    NPAD = -(-N // (NS * 80)) * (NS * 80)
    src = edge_index[0]
    dst = edge_index[1]

    degp = _make_deg_kernel(E, NPAD)(dst)
    scatter = _make_scatter_kernel(E, NPAD, CW)

    g1 = _tc_encode(x, W1, b1, Wg1, degp)
    s1 = scatter(g1.reshape(N * NCH, CW), src, dst)
    g2 = _tc_mid(s1, g1, degp, bg1, Wg2)
    s2 = scatter(g2.reshape(N * NCH, CW), src, dst)
    loss, logits = _tc_head(s2, g2, degp, bg2, batch, sact_cids, Wh, bh)
    return loss.reshape(()), logits


# speedup vs baseline: 4.4679x; 1.3815x over previous
import functools

import jax
import jax.numpy as jnp
from jax import lax
from jax.experimental import pallas as pl
from jax.experimental.pallas import tpu as pltpu
from jax.experimental.pallas import tpu_sc as plsc

NC = 2
NS = 16
NCH = 8


@functools.lru_cache(maxsize=None)
def _make_deg_kernel(E: int, NPAD: int):
    EPT = E // (NC * NS)
    K = 40
    NB = EPT // K
    RPT = NPAD // NS
    ZR = 80
    assert EPT % K == 0 and RPT % ZR == 0 and RPT % 8 == 0
    mesh = plsc.VectorSubcoreMesh(core_axis_name="c", subcore_axis_name="s")

    def body(dst_hbm, out_hbm, dst_v, ones_v, zeros_v, acc):
        cid = lax.axis_index("c")
        sid = lax.axis_index("s")
        for r in range(K):
            ones_v[r] = jnp.ones((16,), jnp.float32)
        for r in range(ZR):
            zeros_v[r] = jnp.zeros((16,), jnp.float32)
        for z in range(RPT // ZR):
            pltpu.sync_copy(zeros_v, acc.at[pl.ds(sid * RPT + z * ZR, ZR)])
        plsc.subcore_barrier()
        ebase = cid * (E // NC) + sid * EPT

        def batch(b, carry):
            pltpu.sync_copy(dst_hbm.at[pl.ds(ebase + b * K, K)], dst_v)
            pltpu.sync_copy(ones_v, acc.at[dst_v], add=True)
            return carry

        lax.fori_loop(0, NB, batch, 0)
        plsc.subcore_barrier()
        pltpu.sync_copy(acc.at[pl.ds(sid * RPT, RPT)],
                        out_hbm.at[cid, pl.ds(sid * RPT, RPT)])

    return pl.kernel(
        body,
        out_type=jax.ShapeDtypeStruct((NC, NPAD, 16), jnp.float32),
        mesh=mesh,
        scratch_types=[
            pltpu.VMEM((K,), jnp.int32),
            pltpu.VMEM((K, 16), jnp.float32),
            pltpu.VMEM((ZR, 16), jnp.float32),
            pltpu.VMEM_SHARED((NPAD, 16), jnp.float32),
        ],
    )


@functools.lru_cache(maxsize=None)
def _make_scatter_kernel(E: int, NPAD: int, CW: int):
    EPT = E // NS
    K = 80
    NB = EPT // K
    RPT = NPAD // NS
    ZR = 80
    CPS = NCH // NC
    assert EPT % K == 0 and RPT % ZR == 0 and K % 16 == 0
    mesh = plsc.VectorSubcoreMesh(core_axis_name="c", subcore_axis_name="s")

    def body(g_hbm, src_hbm, dst_hbm, out_hbm,
             srcs_v, dsts_v, dst_v, gidx_v, rows_v, zeros_v, acc):
        cid = lax.axis_index("c")
        sid = lax.axis_index("s")

        def fill_zero(r, carry):
            for i in range(CW // 16):
                zeros_v[r, pl.ds(i * 16, 16)] = jnp.zeros((16,), jnp.float32)
            return carry

        lax.fori_loop(0, ZR, fill_zero, 0)
        pltpu.sync_copy(src_hbm.at[pl.ds(sid * EPT, EPT)], srcs_v)
        pltpu.sync_copy(dst_hbm.at[pl.ds(sid * EPT, EPT)], dsts_v)
        for j in range(CPS):
            chunk = cid * CPS + j
            for z in range(RPT // ZR):
                pltpu.sync_copy(zeros_v, acc.at[pl.ds(sid * RPT + z * ZR, ZR)])
            plsc.subcore_barrier()

            def batch(b, carry):
                for i in range(K // 16):
                    gidx_v[pl.ds(i * 16, 16)] = (
                        srcs_v[pl.ds(b * K + i * 16, 16)] * NCH + chunk)
                    dst_v[pl.ds(i * 16, 16)] = dsts_v[pl.ds(b * K + i * 16, 16)]
                pltpu.sync_copy(g_hbm.at[gidx_v], rows_v)
                pltpu.sync_copy(rows_v, acc.at[dst_v], add=True)
                return carry

            lax.fori_loop(0, NB, batch, 0)
            plsc.subcore_barrier()
            pltpu.sync_copy(acc.at[pl.ds(sid * RPT, RPT)],
                            out_hbm.at[chunk, pl.ds(sid * RPT, RPT)])

    return pl.kernel(
        body,
        out_type=jax.ShapeDtypeStruct((NCH, NPAD, CW), jnp.float32),
        mesh=mesh,
        scratch_types=[
            pltpu.VMEM((E // NS,), jnp.int32),
            pltpu.VMEM((E // NS,), jnp.int32),
            pltpu.VMEM((K,), jnp.int32),
            pltpu.VMEM((K,), jnp.int32),
            pltpu.VMEM((K, CW), jnp.float32),
            pltpu.VMEM((ZR, CW), jnp.float32),
            pltpu.VMEM_SHARED((NPAD, CW), jnp.float32),
        ],
    )


_R = 1000


def _dis_from_parts(dp):
    deg = dp[0, :, 0:1] + dp[1, :, 0:1] + 1.0
    return lax.rsqrt(deg)


def _encode_body(x_ref, w1_ref, b1_ref, wg1_ref, degp_ref, out_ref):
    a = jnp.dot(x_ref[...], w1_ref[...], preferred_element_type=jnp.float32)
    a = jnp.maximum(a + b1_ref[...], 0.0)
    t = jnp.dot(a, wg1_ref[...], preferred_element_type=jnp.float32)
    out_ref[...] = _dis_from_parts(degp_ref[...]) * t


def _tc_encode(x, W1, b1, Wg1, degp):
    N, DIN = x.shape
    D = Wg1.shape[0]
    grid = (N // _R,)
    return pl.pallas_call(
        _encode_body,
        grid=grid,
        in_specs=[
            pl.BlockSpec((_R, DIN), lambda i: (i, 0)),
            pl.BlockSpec((DIN, D), lambda i: (0, 0)),
            pl.BlockSpec((1, D), lambda i: (0, 0)),
            pl.BlockSpec((D, D), lambda i: (0, 0)),
            pl.BlockSpec((NC, _R, 16), lambda i: (0, i, 0)),
        ],
        out_specs=pl.BlockSpec((_R, D), lambda i: (i, 0)),
        out_shape=jax.ShapeDtypeStruct((N, D), jnp.float32),
    )(x, W1, b1.reshape(1, D), Wg1, degp)


def _combine(s, g, dis, bg):
    scat = jnp.concatenate([s[c] for c in range(NCH)], axis=1)
    return jnp.maximum(dis * (scat + g) + bg, 0.0)


def _mid_body(s_ref, g_ref, degp_ref, bg_ref, w_ref, out_ref):
    dis = _dis_from_parts(degp_ref[...])
    h = _combine(s_ref[...], g_ref[...], dis, bg_ref[...])
    t = jnp.dot(h, w_ref[...], preferred_element_type=jnp.float32)
    out_ref[...] = dis * t


def _tc_mid(s_cm, g, degp, bg, W):
    N, D = g.shape
    CW = D // NCH
    grid = (N // _R,)
    return pl.pallas_call(
        _mid_body,
        grid=grid,
        in_specs=[
            pl.BlockSpec((NCH, _R, CW), lambda i: (0, i, 0)),
            pl.BlockSpec((_R, D), lambda i: (i, 0)),
            pl.BlockSpec((NC, _R, 16), lambda i: (0, i, 0)),
            pl.BlockSpec((1, D), lambda i: (0, 0)),
            pl.BlockSpec((D, D), lambda i: (0, 0)),
        ],
        out_specs=pl.BlockSpec((_R, D), lambda i: (i, 0)),
        out_shape=jax.ShapeDtypeStruct((N, D), jnp.float32),
    )(s_cm, g, degp, bg.reshape(1, D), W)


def _head_body(s_ref, g_ref, degp_ref, bg_ref, batch_ref, cids_ref, wh_ref,
               bh_ref, loss_ref, logits_ref, pooled_acc, cnt_acc, *, nsteps, nb, nc):
    i = pl.program_id(0)

    @pl.when(i == 0)
    def _init():
        pooled_acc[...] = jnp.zeros_like(pooled_acc)
        cnt_acc[...] = jnp.zeros_like(cnt_acc)

    dis = _dis_from_parts(degp_ref[...])
    h = _combine(s_ref[...], g_ref[...], dis, bg_ref[...])
    bcol = batch_ref[...]
    mask = (bcol == lax.broadcasted_iota(jnp.int32, (_R, nb), 1)
            ).astype(jnp.float32)
    pooled_acc[...] += lax.dot_general(
        mask, h, (((0,), (0,)), ((), ())), preferred_element_type=jnp.float32)
    cnt_acc[...] += jnp.sum(mask, axis=0)[:, None]

    @pl.when(i == nsteps - 1)
    def _fin():
        pooled = pooled_acc[...] / jnp.maximum(cnt_acc[...], 1.0)
        logits = jnp.dot(pooled, wh_ref[...],
                         preferred_element_type=jnp.float32) + bh_ref[...]
        logits_ref[...] = logits
        m = jnp.max(logits, axis=1, keepdims=True)
        lse = jnp.log(jnp.sum(jnp.exp(logits - m), axis=1, keepdims=True)) + m
        logp = logits - lse
        oh = (cids_ref[...] == lax.broadcasted_iota(jnp.int32, (nb, nc), 1)
              ).astype(jnp.float32)
        loss_ref[...] = (-jnp.sum(logp * oh) / nb).reshape(1, 1)


def _tc_head(s_cm, g, degp, bg, batch, cids, Wh, bh):
    N, D = g.shape
    CW = D // NCH
    C = Wh.shape[1]
    B = cids.shape[0]
    nsteps = N // _R
    body = functools.partial(_head_body, nsteps=nsteps, nb=B, nc=C)
    return pl.pallas_call(
        body,
        grid=(nsteps,),
        in_specs=[
            pl.BlockSpec((NCH, _R, CW), lambda i: (0, i, 0)),
            pl.BlockSpec((_R, D), lambda i: (i, 0)),
            pl.BlockSpec((NC, _R, 16), lambda i: (0, i, 0)),
            pl.BlockSpec((1, D), lambda i: (0, 0)),
            pl.BlockSpec((_R, 1), lambda i: (i, 0)),
            pl.BlockSpec((B, 1), lambda i: (0, 0)),
            pl.BlockSpec((D, C), lambda i: (0, 0)),
            pl.BlockSpec((1, C), lambda i: (0, 0)),
        ],
        out_specs=[
            pl.BlockSpec((1, 1), lambda i: (0, 0)),
            pl.BlockSpec((B, C), lambda i: (0, 0)),
        ],
        out_shape=[
            jax.ShapeDtypeStruct((1, 1), jnp.float32),
            jax.ShapeDtypeStruct((B, C), jnp.float32),
        ],
        scratch_shapes=[
            pltpu.VMEM((B, D), jnp.float32),
            pltpu.VMEM((B, 1), jnp.float32),
        ],
    )(s_cm, g, degp, bg.reshape(1, D), batch.reshape(N, 1),
      cids.reshape(B, 1), Wh, bh.reshape(1, C))


def kernel(x, edge_index, batch, sact_cids, W1, b1, Wg1, bg1, Wg2, bg2, Wh, bh):
    N = x.shape[0]
    E = edge_index.shape[1]
    D = Wg1.shape[0]
    CW = D // NCH
    NPAD = -(-N // (NS * 80)) * (NS * 80)
    src = edge_index[0]
    dst = edge_index[1]
    EPAD = -(-E // (NS * 80 * 8)) * (NS * 80 * 8)
    npe = EPAD - E
    if npe:
        assert NPAD > N
        ar = jnp.arange(npe, dtype=jnp.int32)
        src = jnp.concatenate([src, ar % N])
        dst = jnp.concatenate([dst, N + ar % (NPAD - N)])

    degp = _make_deg_kernel(EPAD, NPAD)(dst)
    scatter = _make_scatter_kernel(EPAD, NPAD, CW)
    src2 = src
    dst2 = dst

    g1 = _tc_encode(x, W1, b1, Wg1, degp)
    s1 = scatter(g1.reshape(N * NCH, CW), src2, dst2)
    g2 = _tc_mid(s1, g1, degp, bg1, Wg2)
    s2 = scatter(g2.reshape(N * NCH, CW), src2, dst2)
    loss, logits = _tc_head(s2, g2, degp, bg2, batch, sact_cids, Wh, bh)
    return loss.reshape(()), logits


# speedup vs baseline: 5.2286x; 1.1703x over previous
import functools

import jax
import jax.numpy as jnp
from jax import lax
from jax.experimental import pallas as pl
from jax.experimental.pallas import tpu as pltpu
from jax.experimental.pallas import tpu_sc as plsc

NC = 2
NS = 16
NCH = 8


@functools.lru_cache(maxsize=None)
def _make_deg_kernel(E: int, NPAD: int):
    EPT = E // (NC * NS)
    K = 40
    NB = EPT // K
    RPT = NPAD // NS
    ZR = 128
    assert EPT % K == 0 and RPT % ZR == 0 and RPT % 8 == 0
    mesh = plsc.VectorSubcoreMesh(core_axis_name="c", subcore_axis_name="s")

    def body(dst_hbm, out_hbm, dst_v, ones_v, zeros_v, acc):
        cid = lax.axis_index("c")
        sid = lax.axis_index("s")
        for r in range(K):
            ones_v[r] = jnp.ones((16,), jnp.float32)
        for r in range(ZR):
            zeros_v[r] = jnp.zeros((16,), jnp.float32)
        for z in range(RPT // ZR):
            pltpu.sync_copy(zeros_v, acc.at[pl.ds(sid * RPT + z * ZR, ZR)])
        plsc.subcore_barrier()
        ebase = cid * (E // NC) + sid * EPT

        def batch(b, carry):
            pltpu.sync_copy(dst_hbm.at[pl.ds(ebase + b * K, K)], dst_v)
            pltpu.sync_copy(ones_v, acc.at[dst_v], add=True)
            return carry

        lax.fori_loop(0, NB, batch, 0)
        plsc.subcore_barrier()
        pltpu.sync_copy(acc.at[pl.ds(sid * RPT, RPT)],
                        out_hbm.at[cid, pl.ds(sid * RPT, RPT)])

    return pl.kernel(
        body,
        out_type=jax.ShapeDtypeStruct((NC, NPAD, 16), jnp.float32),
        mesh=mesh,
        scratch_types=[
            pltpu.VMEM((K,), jnp.int32),
            pltpu.VMEM((K, 16), jnp.float32),
            pltpu.VMEM((ZR, 16), jnp.float32),
            pltpu.VMEM_SHARED((NPAD, 16), jnp.float32),
        ],
    )


@functools.lru_cache(maxsize=None)
def _make_scatter_kernel(E: int, NPAD: int, CW: int):
    EPT = E // NS
    K = 80
    NB = EPT // K
    RPT = NPAD // NS
    ZR = 128
    CPS = NCH // NC
    assert EPT % K == 0 and RPT % ZR == 0 and K % 16 == 0 and NB % 2 == 1
    mesh = plsc.VectorSubcoreMesh(core_axis_name="c", subcore_axis_name="s")

    def body(g_hbm, src_hbm, dst_hbm, out_hbm,
             src0, dst0, gidx0, rows0, src1, dst1, gidx1, rows1,
             zeros_v, acc, isem0, gsem0, ssem0, isem1, gsem1, ssem1):
        cid = lax.axis_index("c")
        sid = lax.axis_index("s")
        slots = ((src0, dst0, gidx0, rows0, isem0, gsem0, ssem0),
                 (src1, dst1, gidx1, rows1, isem1, gsem1, ssem1))

        def fill_zero(r, carry):
            for i in range(CW // 16):
                zeros_v[r, pl.ds(i * 16, 16)] = jnp.zeros((16,), jnp.float32)
            return carry

        lax.fori_loop(0, ZR, fill_zero, 0)
        ebase = sid * EPT

        def idx_start(bat, slot):
            sv, dv, gv, rv, isem, gsem, ssem = slots[slot]
            off = ebase + bat * K
            pltpu.async_copy(src_hbm.at[pl.ds(off, K)], sv, isem)
            pltpu.async_copy(dst_hbm.at[pl.ds(off, K)], dv, isem)

        def idx_wait(bat, slot):
            sv, dv, gv, rv, isem, gsem, ssem = slots[slot]
            off = ebase + bat * K
            pltpu.make_async_copy(src_hbm.at[pl.ds(off, K)], sv, isem).wait()
            pltpu.make_async_copy(dst_hbm.at[pl.ds(off, K)], dv, isem).wait()

        for j in range(CPS):
            chunk = cid * CPS + j
            for z in range(RPT // ZR):
                pltpu.sync_copy(zeros_v, acc.at[pl.ds(sid * RPT + z * ZR, ZR)])
            plsc.subcore_barrier()

            idx_start(0, 0)
            idx_start(1, 1)

            def pair(gi, carry):
                for s in range(2):
                    bat = gi * 2 + s
                    sv, dv, gv, rv, isem, gsem, ssem = slots[s]
                    idx_wait(bat, s)
                    for i in range(K // 16):
                        gv[pl.ds(i * 16, 16)] = (
                            sv[pl.ds(i * 16, 16)] * NCH + chunk)
                    pltpu.async_copy(g_hbm.at[gv], rv, gsem)
                for s in range(2):
                    sv, dv, gv, rv, isem, gsem, ssem = slots[s]
                    pltpu.make_async_copy(g_hbm.at[gv], rv, gsem).wait()
                    pltpu.async_copy(rv, acc.at[dv], ssem, add=True)
                for s in range(2):
                    bat = gi * 2 + s
                    sv, dv, gv, rv, isem, gsem, ssem = slots[s]
                    pltpu.make_async_copy(rv, acc.at[dv], ssem).wait()

                    @pl.when(bat + 2 < NB)
                    def _pref():
                        idx_start(bat + 2, s)

                return carry

            lax.fori_loop(0, NB // 2, pair, 0)
            bat = NB - 1
            sv, dv, gv, rv, isem, gsem, ssem = slots[0]
            idx_wait(bat, 0)
            for i in range(K // 16):
                gv[pl.ds(i * 16, 16)] = sv[pl.ds(i * 16, 16)] * NCH + chunk
            pltpu.sync_copy(g_hbm.at[gv], rv)
            pltpu.sync_copy(rv, acc.at[dv], add=True)
            plsc.subcore_barrier()
            pltpu.sync_copy(acc.at[pl.ds(sid * RPT, RPT)],
                            out_hbm.at[chunk, pl.ds(sid * RPT, RPT)])

    return pl.kernel(
        body,
        out_type=jax.ShapeDtypeStruct((NCH, NPAD, CW), jnp.float32),
        mesh=mesh,
        scratch_types=[
            pltpu.VMEM((K,), jnp.int32),
            pltpu.VMEM((K,), jnp.int32),
            pltpu.VMEM((K,), jnp.int32),
            pltpu.VMEM((K, CW), jnp.float32),
            pltpu.VMEM((K,), jnp.int32),
            pltpu.VMEM((K,), jnp.int32),
            pltpu.VMEM((K,), jnp.int32),
            pltpu.VMEM((K, CW), jnp.float32),
            pltpu.VMEM((ZR, CW), jnp.float32),
            pltpu.VMEM_SHARED((NPAD, CW), jnp.float32),
            pltpu.SemaphoreType.DMA,
            pltpu.SemaphoreType.DMA,
            pltpu.SemaphoreType.DMA,
            pltpu.SemaphoreType.DMA,
            pltpu.SemaphoreType.DMA,
            pltpu.SemaphoreType.DMA,
        ],
    )


_R = 1000


def _dis_from_parts(dp):
    deg = dp[0, :, 0:1] + dp[1, :, 0:1] + 1.0
    return lax.rsqrt(deg)


def _encode_body(x_ref, w1_ref, b1_ref, wg1_ref, degp_ref, out_ref):
    a = jnp.dot(x_ref[...], w1_ref[...], preferred_element_type=jnp.float32)
    a = jnp.maximum(a + b1_ref[...], 0.0)
    t = jnp.dot(a, wg1_ref[...], preferred_element_type=jnp.float32)
    out_ref[...] = _dis_from_parts(degp_ref[...]) * t


def _tc_encode(x, W1, b1, Wg1, degp):
    N, DIN = x.shape
    D = Wg1.shape[0]
    grid = (N // _R,)
    return pl.pallas_call(
        _encode_body,
        grid=grid,
        in_specs=[
            pl.BlockSpec((_R, DIN), lambda i: (i, 0)),
            pl.BlockSpec((DIN, D), lambda i: (0, 0)),
            pl.BlockSpec((1, D), lambda i: (0, 0)),
            pl.BlockSpec((D, D), lambda i: (0, 0)),
            pl.BlockSpec((NC, _R, 16), lambda i: (0, i, 0)),
        ],
        out_specs=pl.BlockSpec((_R, D), lambda i: (i, 0)),
        out_shape=jax.ShapeDtypeStruct((N, D), jnp.float32),
    )(x, W1, b1.reshape(1, D), Wg1, degp)


def _combine(s, g, dis, bg):
    scat = jnp.concatenate([s[c] for c in range(NCH)], axis=1)
    return jnp.maximum(dis * (scat + g) + bg, 0.0)


def _mid_body(s_ref, g_ref, degp_ref, bg_ref, w_ref, out_ref):
    dis = _dis_from_parts(degp_ref[...])
    h = _combine(s_ref[...], g_ref[...], dis, bg_ref[...])
    t = jnp.dot(h, w_ref[...], preferred_element_type=jnp.float32)
    out_ref[...] = dis * t


def _tc_mid(s_cm, g, degp, bg, W):
    N, D = g.shape
    CW = D // NCH
    grid = (N // _R,)
    return pl.pallas_call(
        _mid_body,
        grid=grid,
        in_specs=[
            pl.BlockSpec((NCH, _R, CW), lambda i: (0, i, 0)),
            pl.BlockSpec((_R, D), lambda i: (i, 0)),
            pl.BlockSpec((NC, _R, 16), lambda i: (0, i, 0)),
            pl.BlockSpec((1, D), lambda i: (0, 0)),
            pl.BlockSpec((D, D), lambda i: (0, 0)),
        ],
        out_specs=pl.BlockSpec((_R, D), lambda i: (i, 0)),
        out_shape=jax.ShapeDtypeStruct((N, D), jnp.float32),
    )(s_cm, g, degp, bg.reshape(1, D), W)


def _head_body(s_ref, g_ref, degp_ref, bg_ref, batch_ref, cids_ref, wh_ref,
               bh_ref, loss_ref, logits_ref, pooled_acc, cnt_acc, *, nsteps, nb, nc):
    i = pl.program_id(0)

    @pl.when(i == 0)
    def _init():
        pooled_acc[...] = jnp.zeros_like(pooled_acc)
        cnt_acc[...] = jnp.zeros_like(cnt_acc)

    dis = _dis_from_parts(degp_ref[...])
    h = _combine(s_ref[...], g_ref[...], dis, bg_ref[...])
    bcol = batch_ref[...]
    mask = (bcol == lax.broadcasted_iota(jnp.int32, (_R, nb), 1)
            ).astype(jnp.float32)
    pooled_acc[...] += lax.dot_general(
        mask, h, (((0,), (0,)), ((), ())), preferred_element_type=jnp.float32)
    cnt_acc[...] += jnp.sum(mask, axis=0)[:, None]

    @pl.when(i == nsteps - 1)
    def _fin():
        pooled = pooled_acc[...] / jnp.maximum(cnt_acc[...], 1.0)
        logits = jnp.dot(pooled, wh_ref[...],
                         preferred_element_type=jnp.float32) + bh_ref[...]
        logits_ref[...] = logits
        m = jnp.max(logits, axis=1, keepdims=True)
        lse = jnp.log(jnp.sum(jnp.exp(logits - m), axis=1, keepdims=True)) + m
        logp = logits - lse
        oh = (cids_ref[...] == lax.broadcasted_iota(jnp.int32, (nb, nc), 1)
              ).astype(jnp.float32)
        loss_ref[...] = (-jnp.sum(logp * oh) / nb).reshape(1, 1)


def _tc_head(s_cm, g, degp, bg, batch, cids, Wh, bh):
    N, D = g.shape
    CW = D // NCH
    C = Wh.shape[1]
    B = cids.shape[0]
    nsteps = N // _R
    body = functools.partial(_head_body, nsteps=nsteps, nb=B, nc=C)
    return pl.pallas_call(
        body,
        grid=(nsteps,),
        in_specs=[
            pl.BlockSpec((NCH, _R, CW), lambda i: (0, i, 0)),
            pl.BlockSpec((_R, D), lambda i: (i, 0)),
            pl.BlockSpec((NC, _R, 16), lambda i: (0, i, 0)),
            pl.BlockSpec((1, D), lambda i: (0, 0)),
            pl.BlockSpec((_R, 1), lambda i: (i, 0)),
            pl.BlockSpec((B, 1), lambda i: (0, 0)),
            pl.BlockSpec((D, C), lambda i: (0, 0)),
            pl.BlockSpec((1, C), lambda i: (0, 0)),
        ],
        out_specs=[
            pl.BlockSpec((1, 1), lambda i: (0, 0)),
            pl.BlockSpec((B, C), lambda i: (0, 0)),
        ],
        out_shape=[
            jax.ShapeDtypeStruct((1, 1), jnp.float32),
            jax.ShapeDtypeStruct((B, C), jnp.float32),
        ],
        scratch_shapes=[
            pltpu.VMEM((B, D), jnp.float32),
            pltpu.VMEM((B, 1), jnp.float32),
        ],
    )(s_cm, g, degp, bg.reshape(1, D), batch.reshape(N, 1),
      cids.reshape(B, 1), Wh, bh.reshape(1, C))


def kernel(x, edge_index, batch, sact_cids, W1, b1, Wg1, bg1, Wg2, bg2, Wh, bh):
    N = x.shape[0]
    E = edge_index.shape[1]
    D = Wg1.shape[0]
    CW = D // NCH
    NPAD = -(-N // (NS * 80)) * (NS * 80)
    src = edge_index[0]
    dst = edge_index[1]
    EPAD = E
    npe = EPAD - E
    if npe:
        assert NPAD > N
        ar = jnp.arange(npe, dtype=jnp.int32)
        src = jnp.concatenate([src, ar % N])
        dst = jnp.concatenate([dst, N + ar % (NPAD - N)])

    degp = _make_deg_kernel(EPAD, NPAD)(dst)
    scatter = _make_scatter_kernel(EPAD, NPAD, CW)
    src2 = src
    dst2 = dst

    g1 = _tc_encode(x, W1, b1, Wg1, degp)
    s1 = scatter(g1.reshape(N * NCH, CW), src2, dst2)
    g2 = _tc_mid(s1, g1, degp, bg1, Wg2)
    s2 = scatter(g2.reshape(N * NCH, CW), src2, dst2)
    loss, logits = _tc_head(s2, g2, degp, bg2, batch, sact_cids, Wh, bh)
    return loss.reshape(()), logits


# speedup vs baseline: 6.2788x; 1.2008x over previous
import functools

import jax
import jax.numpy as jnp
from jax import lax
from jax.experimental import pallas as pl
from jax.experimental.pallas import tpu as pltpu
from jax.experimental.pallas import tpu_sc as plsc

NC = 2
NS = 16
NCH = 8


@functools.lru_cache(maxsize=None)
def _make_deg_kernel(E: int, NPAD: int):
    EPT = E // (NC * NS)
    K = 40
    NB = EPT // K
    RPT = NPAD // NS
    ZR = 128
    assert EPT % K == 0 and RPT % ZR == 0 and RPT % 8 == 0
    mesh = plsc.VectorSubcoreMesh(core_axis_name="c", subcore_axis_name="s")

    def body(dst_hbm, out_hbm, dst_v, ones_v, zeros_v, acc):
        cid = lax.axis_index("c")
        sid = lax.axis_index("s")
        for r in range(K):
            ones_v[r] = jnp.ones((16,), jnp.float32)
        for r in range(ZR):
            zeros_v[r] = jnp.zeros((16,), jnp.float32)
        for z in range(RPT // ZR):
            pltpu.sync_copy(zeros_v, acc.at[pl.ds(sid * RPT + z * ZR, ZR)])
        plsc.subcore_barrier()
        ebase = cid * (E // NC) + sid * EPT

        def batch(b, carry):
            pltpu.sync_copy(dst_hbm.at[pl.ds(ebase + b * K, K)], dst_v)
            pltpu.sync_copy(ones_v, acc.at[dst_v], add=True)
            return carry

        lax.fori_loop(0, NB, batch, 0)
        plsc.subcore_barrier()
        pltpu.sync_copy(acc.at[pl.ds(sid * RPT, RPT)],
                        out_hbm.at[cid, pl.ds(sid * RPT, RPT)])

    return pl.kernel(
        body,
        out_type=jax.ShapeDtypeStruct((NC, NPAD, 16), jnp.float32),
        mesh=mesh,
        scratch_types=[
            pltpu.VMEM((K,), jnp.int32),
            pltpu.VMEM((K, 16), jnp.float32),
            pltpu.VMEM((ZR, 16), jnp.float32),
            pltpu.VMEM_SHARED((NPAD, 16), jnp.float32),
        ],
    )


@functools.lru_cache(maxsize=None)
def _make_scatter_kernel(E: int, NPAD: int, CW: int):
    EPT = E // NS
    K = 80
    NB = EPT // K
    RPT = NPAD // NS
    ZR = 128
    CPS = NCH // NC
    NSLOT = 3
    NG = NB // NSLOT
    TAIL = NB - NG * NSLOT
    assert EPT % K == 0 and RPT % ZR == 0 and K % 16 == 0 and TAIL < NSLOT
    mesh = plsc.VectorSubcoreMesh(core_axis_name="c", subcore_axis_name="s")

    def body(g_hbm, src_hbm, dst_hbm, out_hbm,
             src0, dst0, gidx0, rows0, src1, dst1, gidx1, rows1,
             src2, dst2, gidx2, rows2, zeros_v, acc,
             isem0, gsem0, ssem0, isem1, gsem1, ssem1, isem2, gsem2, ssem2):
        cid = lax.axis_index("c")
        sid = lax.axis_index("s")
        slots = ((src0, dst0, gidx0, rows0, isem0, gsem0, ssem0),
                 (src1, dst1, gidx1, rows1, isem1, gsem1, ssem1),
                 (src2, dst2, gidx2, rows2, isem2, gsem2, ssem2))

        def fill_zero(r, carry):
            for i in range(CW // 16):
                zeros_v[r, pl.ds(i * 16, 16)] = jnp.zeros((16,), jnp.float32)
            return carry

        lax.fori_loop(0, ZR, fill_zero, 0)
        ebase = sid * EPT

        def idx_start(bat, slot):
            sv, dv, gv, rv, isem, gsem, ssem = slots[slot]
            off = ebase + bat * K
            pltpu.async_copy(src_hbm.at[pl.ds(off, K)], sv, isem)
            pltpu.async_copy(dst_hbm.at[pl.ds(off, K)], dv, isem)

        def idx_wait(bat, slot):
            sv, dv, gv, rv, isem, gsem, ssem = slots[slot]
            off = ebase + bat * K
            pltpu.make_async_copy(src_hbm.at[pl.ds(off, K)], sv, isem).wait()
            pltpu.make_async_copy(dst_hbm.at[pl.ds(off, K)], dv, isem).wait()

        for j in range(CPS):
            chunk = cid * CPS + j
            for z in range(RPT // ZR):
                pltpu.sync_copy(zeros_v, acc.at[pl.ds(sid * RPT + z * ZR, ZR)])
            plsc.subcore_barrier()

            for s in range(NSLOT):
                idx_start(s, s)

            def group(gi, carry):
                for s in range(NSLOT):
                    bat = gi * NSLOT + s
                    sv, dv, gv, rv, isem, gsem, ssem = slots[s]
                    idx_wait(bat, s)
                    for i in range(K // 16):
                        gv[pl.ds(i * 16, 16)] = (
                            sv[pl.ds(i * 16, 16)] * NCH + chunk)
                    pltpu.async_copy(g_hbm.at[gv], rv, gsem)
                for s in range(NSLOT):
                    sv, dv, gv, rv, isem, gsem, ssem = slots[s]
                    pltpu.make_async_copy(g_hbm.at[gv], rv, gsem).wait()
                    pltpu.async_copy(rv, acc.at[dv], ssem, add=True)
                for s in range(NSLOT):
                    bat = gi * NSLOT + s
                    sv, dv, gv, rv, isem, gsem, ssem = slots[s]
                    pltpu.make_async_copy(rv, acc.at[dv], ssem).wait()

                    @pl.when(bat + NSLOT < NB)
                    def _pref():
                        idx_start(bat + NSLOT, s)

                return carry

            lax.fori_loop(0, NG, group, 0)
            for s in range(TAIL):
                bat = NG * NSLOT + s
                sv, dv, gv, rv, isem, gsem, ssem = slots[s]
                idx_wait(bat, s)
                for i in range(K // 16):
                    gv[pl.ds(i * 16, 16)] = sv[pl.ds(i * 16, 16)] * NCH + chunk
                pltpu.sync_copy(g_hbm.at[gv], rv)
                pltpu.sync_copy(rv, acc.at[dv], add=True)
            plsc.subcore_barrier()
            pltpu.sync_copy(acc.at[pl.ds(sid * RPT, RPT)],
                            out_hbm.at[chunk, pl.ds(sid * RPT, RPT)])

    return pl.kernel(
        body,
        out_type=jax.ShapeDtypeStruct((NCH, NPAD, CW), jnp.float32),
        mesh=mesh,
        scratch_types=[
            pltpu.VMEM((K,), jnp.int32),
            pltpu.VMEM((K,), jnp.int32),
            pltpu.VMEM((K,), jnp.int32),
            pltpu.VMEM((K, CW), jnp.float32),
            pltpu.VMEM((K,), jnp.int32),
            pltpu.VMEM((K,), jnp.int32),
            pltpu.VMEM((K,), jnp.int32),
            pltpu.VMEM((K, CW), jnp.float32),
            pltpu.VMEM((K,), jnp.int32),
            pltpu.VMEM((K,), jnp.int32),
            pltpu.VMEM((K,), jnp.int32),
            pltpu.VMEM((K, CW), jnp.float32),
            pltpu.VMEM((ZR, CW), jnp.float32),
            pltpu.VMEM_SHARED((NPAD, CW), jnp.float32),
            pltpu.SemaphoreType.DMA,
            pltpu.SemaphoreType.DMA,
            pltpu.SemaphoreType.DMA,
            pltpu.SemaphoreType.DMA,
            pltpu.SemaphoreType.DMA,
            pltpu.SemaphoreType.DMA,
            pltpu.SemaphoreType.DMA,
            pltpu.SemaphoreType.DMA,
            pltpu.SemaphoreType.DMA,
        ],
    )


_R = 1000


def _dis_from_parts(dp):
    deg = dp[0, :, 0:1] + dp[1, :, 0:1] + 1.0
    return lax.rsqrt(deg)


def _encode_body(x_ref, w1_ref, b1_ref, wg1_ref, degp_ref, out_ref):
    a = jnp.dot(x_ref[...], w1_ref[...], preferred_element_type=jnp.float32)
    a = jnp.maximum(a + b1_ref[...], 0.0)
    t = jnp.dot(a, wg1_ref[...], preferred_element_type=jnp.float32)
    out_ref[...] = _dis_from_parts(degp_ref[...]) * t


def _tc_encode(x, W1, b1, Wg1, degp):
    N, DIN = x.shape
    D = Wg1.shape[0]
    grid = (N // _R,)
    return pl.pallas_call(
        _encode_body,
        grid=grid,
        in_specs=[
            pl.BlockSpec((_R, DIN), lambda i: (i, 0)),
            pl.BlockSpec((DIN, D), lambda i: (0, 0)),
            pl.BlockSpec((1, D), lambda i: (0, 0)),
            pl.BlockSpec((D, D), lambda i: (0, 0)),
            pl.BlockSpec((NC, _R, 16), lambda i: (0, i, 0)),
        ],
        out_specs=pl.BlockSpec((_R, D), lambda i: (i, 0)),
        out_shape=jax.ShapeDtypeStruct((N, D), jnp.float32),
    )(x, W1, b1.reshape(1, D), Wg1, degp)


def _combine(s, g, dis, bg):
    scat = jnp.concatenate([s[c] for c in range(NCH)], axis=1)
    return jnp.maximum(dis * (scat + g) + bg, 0.0)


def _mid_body(s_ref, g_ref, degp_ref, bg_ref, w_ref, out_ref):
    dis = _dis_from_parts(degp_ref[...])
    h = _combine(s_ref[...], g_ref[...], dis, bg_ref[...])
    t = jnp.dot(h, w_ref[...], preferred_element_type=jnp.float32)
    out_ref[...] = dis * t


def _tc_mid(s_cm, g, degp, bg, W):
    N, D = g.shape
    CW = D // NCH
    grid = (N // _R,)
    return pl.pallas_call(
        _mid_body,
        grid=grid,
        in_specs=[
            pl.BlockSpec((NCH, _R, CW), lambda i: (0, i, 0)),
            pl.BlockSpec((_R, D), lambda i: (i, 0)),
            pl.BlockSpec((NC, _R, 16), lambda i: (0, i, 0)),
            pl.BlockSpec((1, D), lambda i: (0, 0)),
            pl.BlockSpec((D, D), lambda i: (0, 0)),
        ],
        out_specs=pl.BlockSpec((_R, D), lambda i: (i, 0)),
        out_shape=jax.ShapeDtypeStruct((N, D), jnp.float32),
    )(s_cm, g, degp, bg.reshape(1, D), W)


def _head_body(s_ref, g_ref, degp_ref, bg_ref, batch_ref, cids_ref, wh_ref,
               bh_ref, loss_ref, logits_ref, pooled_acc, cnt_acc, *, nsteps, nb, nc):
    i = pl.program_id(0)

    @pl.when(i == 0)
    def _init():
        pooled_acc[...] = jnp.zeros_like(pooled_acc)
        cnt_acc[...] = jnp.zeros_like(cnt_acc)

    dis = _dis_from_parts(degp_ref[...])
    h = _combine(s_ref[...], g_ref[...], dis, bg_ref[...])
    bcol = batch_ref[...]
    mask = (bcol == lax.broadcasted_iota(jnp.int32, (_R, nb), 1)
            ).astype(jnp.float32)
    pooled_acc[...] += lax.dot_general(
        mask, h, (((0,), (0,)), ((), ())), preferred_element_type=jnp.float32)
    cnt_acc[...] += jnp.sum(mask, axis=0)[:, None]

    @pl.when(i == nsteps - 1)
    def _fin():
        pooled = pooled_acc[...] / jnp.maximum(cnt_acc[...], 1.0)
        logits = jnp.dot(pooled, wh_ref[...],
                         preferred_element_type=jnp.float32) + bh_ref[...]
        logits_ref[...] = logits
        m = jnp.max(logits, axis=1, keepdims=True)
        lse = jnp.log(jnp.sum(jnp.exp(logits - m), axis=1, keepdims=True)) + m
        logp = logits - lse
        oh = (cids_ref[...] == lax.broadcasted_iota(jnp.int32, (nb, nc), 1)
              ).astype(jnp.float32)
        loss_ref[...] = (-jnp.sum(logp * oh) / nb).reshape(1, 1)


def _tc_head(s_cm, g, degp, bg, batch, cids, Wh, bh):
    N, D = g.shape
    CW = D // NCH
    C = Wh.shape[1]
    B = cids.shape[0]
    nsteps = N // _R
    body = functools.partial(_head_body, nsteps=nsteps, nb=B, nc=C)
    return pl.pallas_call(
        body,
        grid=(nsteps,),
        in_specs=[
            pl.BlockSpec((NCH, _R, CW), lambda i: (0, i, 0)),
            pl.BlockSpec((_R, D), lambda i: (i, 0)),
            pl.BlockSpec((NC, _R, 16), lambda i: (0, i, 0)),
            pl.BlockSpec((1, D), lambda i: (0, 0)),
            pl.BlockSpec((_R, 1), lambda i: (i, 0)),
            pl.BlockSpec((B, 1), lambda i: (0, 0)),
            pl.BlockSpec((D, C), lambda i: (0, 0)),
            pl.BlockSpec((1, C), lambda i: (0, 0)),
        ],
        out_specs=[
            pl.BlockSpec((1, 1), lambda i: (0, 0)),
            pl.BlockSpec((B, C), lambda i: (0, 0)),
        ],
        out_shape=[
            jax.ShapeDtypeStruct((1, 1), jnp.float32),
            jax.ShapeDtypeStruct((B, C), jnp.float32),
        ],
        scratch_shapes=[
            pltpu.VMEM((B, D), jnp.float32),
            pltpu.VMEM((B, 1), jnp.float32),
        ],
    )(s_cm, g, degp, bg.reshape(1, D), batch.reshape(N, 1),
      cids.reshape(B, 1), Wh, bh.reshape(1, C))


def kernel(x, edge_index, batch, sact_cids, W1, b1, Wg1, bg1, Wg2, bg2, Wh, bh):
    N = x.shape[0]
    E = edge_index.shape[1]
    D = Wg1.shape[0]
    CW = D // NCH
    NPAD = -(-N // (NS * 80)) * (NS * 80)
    src = edge_index[0]
    dst = edge_index[1]
    EPAD = E
    npe = EPAD - E
    if npe:
        assert NPAD > N
        ar = jnp.arange(npe, dtype=jnp.int32)
        src = jnp.concatenate([src, ar % N])
        dst = jnp.concatenate([dst, N + ar % (NPAD - N)])

    degp = _make_deg_kernel(EPAD, NPAD)(dst)
    scatter = _make_scatter_kernel(EPAD, NPAD, CW)
    src2 = src
    dst2 = dst

    g1 = _tc_encode(x, W1, b1, Wg1, degp)
    s1 = scatter(g1.reshape(N * NCH, CW), src2, dst2)
    g2 = _tc_mid(s1, g1, degp, bg1, Wg2)
    s2 = scatter(g2.reshape(N * NCH, CW), src2, dst2)
    loss, logits = _tc_head(s2, g2, degp, bg2, batch, sact_cids, Wh, bh)
    return loss.reshape(()), logits


# speedup vs baseline: 6.4597x; 1.0288x over previous
import functools

import jax
import jax.numpy as jnp
from jax import lax
from jax.experimental import pallas as pl
from jax.experimental.pallas import tpu as pltpu
from jax.experimental.pallas import tpu_sc as plsc

NC = 2
NS = 16
NCH = 8


@functools.lru_cache(maxsize=None)
def _make_deg_kernel(E: int, NPAD: int):
    EPT = E // (NC * NS)
    K = 40
    NB = EPT // K
    NSLOT = 3
    NG = NB // NSLOT
    TAIL = NB - NG * NSLOT
    RPT = NPAD // NS
    ZR = 128
    assert EPT % K == 0 and RPT % ZR == 0 and RPT % 8 == 0 and TAIL < NSLOT
    mesh = plsc.VectorSubcoreMesh(core_axis_name="c", subcore_axis_name="s")

    def body(dst_hbm, out_hbm, dst0, dst1, dst2, ones_v, zeros_v, acc,
             isem0, ssem0, isem1, ssem1, isem2, ssem2):
        cid = lax.axis_index("c")
        sid = lax.axis_index("s")
        slots = ((dst0, isem0, ssem0), (dst1, isem1, ssem1),
                 (dst2, isem2, ssem2))
        for r in range(K):
            ones_v[r] = jnp.ones((16,), jnp.float32)
        for r in range(ZR):
            zeros_v[r] = jnp.zeros((16,), jnp.float32)
        for z in range(RPT // ZR):
            pltpu.sync_copy(zeros_v, acc.at[pl.ds(sid * RPT + z * ZR, ZR)])
        plsc.subcore_barrier()
        ebase = cid * (E // NC) + sid * EPT

        def idx_start(bat, slot):
            dv, isem, ssem = slots[slot]
            pltpu.async_copy(dst_hbm.at[pl.ds(ebase + bat * K, K)], dv, isem)

        def idx_wait(bat, slot):
            dv, isem, ssem = slots[slot]
            pltpu.make_async_copy(dst_hbm.at[pl.ds(ebase + bat * K, K)], dv,
                                  isem).wait()

        for s in range(NSLOT):
            idx_start(s, s)

        def group(gi, carry):
            for s in range(NSLOT):
                bat = gi * NSLOT + s
                dv, isem, ssem = slots[s]
                idx_wait(bat, s)
                pltpu.async_copy(ones_v, acc.at[dv], ssem, add=True)
            for s in range(NSLOT):
                bat = gi * NSLOT + s
                dv, isem, ssem = slots[s]
                pltpu.make_async_copy(ones_v, acc.at[dv], ssem).wait()

                @pl.when(bat + NSLOT < NB)
                def _pref():
                    idx_start(bat + NSLOT, s)

            return carry

        lax.fori_loop(0, NG, group, 0)
        for s in range(TAIL):
            bat = NG * NSLOT + s
            dv, isem, ssem = slots[s]
            idx_wait(bat, s)
            pltpu.sync_copy(ones_v, acc.at[dv], add=True)
        plsc.subcore_barrier()
        pltpu.sync_copy(acc.at[pl.ds(sid * RPT, RPT)],
                        out_hbm.at[cid, pl.ds(sid * RPT, RPT)])

    return pl.kernel(
        body,
        out_type=jax.ShapeDtypeStruct((NC, NPAD, 16), jnp.float32),
        mesh=mesh,
        scratch_types=[
            pltpu.VMEM((K,), jnp.int32),
            pltpu.VMEM((K,), jnp.int32),
            pltpu.VMEM((K,), jnp.int32),
            pltpu.VMEM((K, 16), jnp.float32),
            pltpu.VMEM((ZR, 16), jnp.float32),
            pltpu.VMEM_SHARED((NPAD, 16), jnp.float32),
            pltpu.SemaphoreType.DMA,
            pltpu.SemaphoreType.DMA,
            pltpu.SemaphoreType.DMA,
            pltpu.SemaphoreType.DMA,
            pltpu.SemaphoreType.DMA,
            pltpu.SemaphoreType.DMA,
        ],
    )


@functools.lru_cache(maxsize=None)
def _make_scatter_kernel(E: int, NPAD: int, CW: int):
    EPT = E // NS
    K = 80
    NB = EPT // K
    RPT = NPAD // NS
    ZR = 128
    CPS = NCH // NC
    NSLOT = 3
    NG = NB // NSLOT
    TAIL = NB - NG * NSLOT
    assert EPT % K == 0 and RPT % ZR == 0 and K % 16 == 0 and TAIL < NSLOT
    mesh = plsc.VectorSubcoreMesh(core_axis_name="c", subcore_axis_name="s")

    def body(g_hbm, src_hbm, dst_hbm, out_hbm,
             src0, dst0, gidx0, rows0, src1, dst1, gidx1, rows1,
             src2, dst2, gidx2, rows2, zeros_v, acc,
             isem0, gsem0, ssem0, isem1, gsem1, ssem1, isem2, gsem2, ssem2):
        cid = lax.axis_index("c")
        sid = lax.axis_index("s")
        slots = ((src0, dst0, gidx0, rows0, isem0, gsem0, ssem0),
                 (src1, dst1, gidx1, rows1, isem1, gsem1, ssem1),
                 (src2, dst2, gidx2, rows2, isem2, gsem2, ssem2))

        def fill_zero(r, carry):
            for i in range(CW // 16):
                zeros_v[r, pl.ds(i * 16, 16)] = jnp.zeros((16,), jnp.float32)
            return carry

        lax.fori_loop(0, ZR, fill_zero, 0)
        ebase = sid * EPT

        def idx_start(bat, slot):
            sv, dv, gv, rv, isem, gsem, ssem = slots[slot]
            off = ebase + bat * K
            pltpu.async_copy(src_hbm.at[pl.ds(off, K)], sv, isem)
            pltpu.async_copy(dst_hbm.at[pl.ds(off, K)], dv, isem)

        def idx_wait(bat, slot):
            sv, dv, gv, rv, isem, gsem, ssem = slots[slot]
            off = ebase + bat * K
            pltpu.make_async_copy(src_hbm.at[pl.ds(off, K)], sv, isem).wait()
            pltpu.make_async_copy(dst_hbm.at[pl.ds(off, K)], dv, isem).wait()

        for j in range(CPS):
            chunk = cid * CPS + j
            for z in range(RPT // ZR):
                pltpu.sync_copy(zeros_v, acc.at[pl.ds(sid * RPT + z * ZR, ZR)])
            plsc.subcore_barrier()

            for s in range(NSLOT):
                idx_start(s, s)

            def group(gi, carry):
                for s in range(NSLOT):
                    bat = gi * NSLOT + s
                    sv, dv, gv, rv, isem, gsem, ssem = slots[s]
                    idx_wait(bat, s)
                    for i in range(K // 16):
                        gv[pl.ds(i * 16, 16)] = (
                            sv[pl.ds(i * 16, 16)] * NCH + chunk)
                    pltpu.async_copy(g_hbm.at[gv], rv, gsem)
                for s in range(NSLOT):
                    sv, dv, gv, rv, isem, gsem, ssem = slots[s]
                    pltpu.make_async_copy(g_hbm.at[gv], rv, gsem).wait()
                    pltpu.async_copy(rv, acc.at[dv], ssem, add=True)
                for s in range(NSLOT):
                    bat = gi * NSLOT + s
                    sv, dv, gv, rv, isem, gsem, ssem = slots[s]
                    pltpu.make_async_copy(rv, acc.at[dv], ssem).wait()

                    @pl.when(bat + NSLOT < NB)
                    def _pref():
                        idx_start(bat + NSLOT, s)

                return carry

            lax.fori_loop(0, NG, group, 0)
            for s in range(TAIL):
                bat = NG * NSLOT + s
                sv, dv, gv, rv, isem, gsem, ssem = slots[s]
                idx_wait(bat, s)
                for i in range(K // 16):
                    gv[pl.ds(i * 16, 16)] = sv[pl.ds(i * 16, 16)] * NCH + chunk
                pltpu.sync_copy(g_hbm.at[gv], rv)
                pltpu.sync_copy(rv, acc.at[dv], add=True)
            plsc.subcore_barrier()
            pltpu.sync_copy(acc.at[pl.ds(sid * RPT, RPT)],
                            out_hbm.at[chunk, pl.ds(sid * RPT, RPT)])

    return pl.kernel(
        body,
        out_type=jax.ShapeDtypeStruct((NCH, NPAD, CW), jnp.float32),
        mesh=mesh,
        scratch_types=[
            pltpu.VMEM((K,), jnp.int32),
            pltpu.VMEM((K,), jnp.int32),
            pltpu.VMEM((K,), jnp.int32),
            pltpu.VMEM((K, CW), jnp.float32),
            pltpu.VMEM((K,), jnp.int32),
            pltpu.VMEM((K,), jnp.int32),
            pltpu.VMEM((K,), jnp.int32),
            pltpu.VMEM((K, CW), jnp.float32),
            pltpu.VMEM((K,), jnp.int32),
            pltpu.VMEM((K,), jnp.int32),
            pltpu.VMEM((K,), jnp.int32),
            pltpu.VMEM((K, CW), jnp.float32),
            pltpu.VMEM((ZR, CW), jnp.float32),
            pltpu.VMEM_SHARED((NPAD, CW), jnp.float32),
            pltpu.SemaphoreType.DMA,
            pltpu.SemaphoreType.DMA,
            pltpu.SemaphoreType.DMA,
            pltpu.SemaphoreType.DMA,
            pltpu.SemaphoreType.DMA,
            pltpu.SemaphoreType.DMA,
            pltpu.SemaphoreType.DMA,
            pltpu.SemaphoreType.DMA,
            pltpu.SemaphoreType.DMA,
        ],
    )


_R = 1000


def _dis_from_parts(dp):
    deg = dp[0, :, 0:1] + dp[1, :, 0:1] + 1.0
    return lax.rsqrt(deg)


def _encode_body(x_ref, w1_ref, b1_ref, wg1_ref, degp_ref, out_ref):
    a = jnp.dot(x_ref[...], w1_ref[...], preferred_element_type=jnp.float32)
    a = jnp.maximum(a + b1_ref[...], 0.0)
    t = jnp.dot(a, wg1_ref[...], preferred_element_type=jnp.float32)
    out_ref[...] = _dis_from_parts(degp_ref[...]) * t


def _tc_encode(x, W1, b1, Wg1, degp):
    N, DIN = x.shape
    D = Wg1.shape[0]
    grid = (N // _R,)
    return pl.pallas_call(
        _encode_body,
        grid=grid,
        in_specs=[
            pl.BlockSpec((_R, DIN), lambda i: (i, 0)),
            pl.BlockSpec((DIN, D), lambda i: (0, 0)),
            pl.BlockSpec((1, D), lambda i: (0, 0)),
            pl.BlockSpec((D, D), lambda i: (0, 0)),
            pl.BlockSpec((NC, _R, 16), lambda i: (0, i, 0)),
        ],
        out_specs=pl.BlockSpec((_R, D), lambda i: (i, 0)),
        out_shape=jax.ShapeDtypeStruct((N, D), jnp.float32),
    )(x, W1, b1.reshape(1, D), Wg1, degp)


def _combine(s, g, dis, bg):
    scat = jnp.concatenate([s[c] for c in range(NCH)], axis=1)
    return jnp.maximum(dis * (scat + g) + bg, 0.0)


def _mid_body(s_ref, g_ref, degp_ref, bg_ref, w_ref, out_ref):
    dis = _dis_from_parts(degp_ref[...])
    h = _combine(s_ref[...], g_ref[...], dis, bg_ref[...])
    t = jnp.dot(h, w_ref[...], preferred_element_type=jnp.float32)
    out_ref[...] = dis * t


def _tc_mid(s_cm, g, degp, bg, W):
    N, D = g.shape
    CW = D // NCH
    grid = (N // _R,)
    return pl.pallas_call(
        _mid_body,
        grid=grid,
        in_specs=[
            pl.BlockSpec((NCH, _R, CW), lambda i: (0, i, 0)),
            pl.BlockSpec((_R, D), lambda i: (i, 0)),
            pl.BlockSpec((NC, _R, 16), lambda i: (0, i, 0)),
            pl.BlockSpec((1, D), lambda i: (0, 0)),
            pl.BlockSpec((D, D), lambda i: (0, 0)),
        ],
        out_specs=pl.BlockSpec((_R, D), lambda i: (i, 0)),
        out_shape=jax.ShapeDtypeStruct((N, D), jnp.float32),
    )(s_cm, g, degp, bg.reshape(1, D), W)


def _head_body(s_ref, g_ref, degp_ref, bg_ref, batch_ref, cids_ref, wh_ref,
               bh_ref, loss_ref, logits_ref, pooled_acc, cnt_acc, *, nsteps, nb, nc):
    i = pl.program_id(0)

    @pl.when(i == 0)
    def _init():
        pooled_acc[...] = jnp.zeros_like(pooled_acc)
        cnt_acc[...] = jnp.zeros_like(cnt_acc)

    dis = _dis_from_parts(degp_ref[...])
    h = _combine(s_ref[...], g_ref[...], dis, bg_ref[...])
    bcol = batch_ref[...]
    mask = (bcol == lax.broadcasted_iota(jnp.int32, (_R, nb), 1)
            ).astype(jnp.float32)
    pooled_acc[...] += lax.dot_general(
        mask, h, (((0,), (0,)), ((), ())), preferred_element_type=jnp.float32)
    cnt_acc[...] += jnp.sum(mask, axis=0)[:, None]

    @pl.when(i == nsteps - 1)
    def _fin():
        pooled = pooled_acc[...] / jnp.maximum(cnt_acc[...], 1.0)
        logits = jnp.dot(pooled, wh_ref[...],
                         preferred_element_type=jnp.float32) + bh_ref[...]
        logits_ref[...] = logits
        m = jnp.max(logits, axis=1, keepdims=True)
        lse = jnp.log(jnp.sum(jnp.exp(logits - m), axis=1, keepdims=True)) + m
        logp = logits - lse
        oh = (cids_ref[...] == lax.broadcasted_iota(jnp.int32, (nb, nc), 1)
              ).astype(jnp.float32)
        loss_ref[...] = (-jnp.sum(logp * oh) / nb).reshape(1, 1)


def _tc_head(s_cm, g, degp, bg, batch, cids, Wh, bh):
    N, D = g.shape
    CW = D // NCH
    C = Wh.shape[1]
    B = cids.shape[0]
    nsteps = N // _R
    body = functools.partial(_head_body, nsteps=nsteps, nb=B, nc=C)
    return pl.pallas_call(
        body,
        grid=(nsteps,),
        in_specs=[
            pl.BlockSpec((NCH, _R, CW), lambda i: (0, i, 0)),
            pl.BlockSpec((_R, D), lambda i: (i, 0)),
            pl.BlockSpec((NC, _R, 16), lambda i: (0, i, 0)),
            pl.BlockSpec((1, D), lambda i: (0, 0)),
            pl.BlockSpec((_R, 1), lambda i: (i, 0)),
            pl.BlockSpec((B, 1), lambda i: (0, 0)),
            pl.BlockSpec((D, C), lambda i: (0, 0)),
            pl.BlockSpec((1, C), lambda i: (0, 0)),
        ],
        out_specs=[
            pl.BlockSpec((1, 1), lambda i: (0, 0)),
            pl.BlockSpec((B, C), lambda i: (0, 0)),
        ],
        out_shape=[
            jax.ShapeDtypeStruct((1, 1), jnp.float32),
            jax.ShapeDtypeStruct((B, C), jnp.float32),
        ],
        scratch_shapes=[
            pltpu.VMEM((B, D), jnp.float32),
            pltpu.VMEM((B, 1), jnp.float32),
        ],
    )(s_cm, g, degp, bg.reshape(1, D), batch.reshape(N, 1),
      cids.reshape(B, 1), Wh, bh.reshape(1, C))


def kernel(x, edge_index, batch, sact_cids, W1, b1, Wg1, bg1, Wg2, bg2, Wh, bh):
    N = x.shape[0]
    E = edge_index.shape[1]
    D = Wg1.shape[0]
    CW = D // NCH
    NPAD = -(-N // (NS * 80)) * (NS * 80)
    src = edge_index[0]
    dst = edge_index[1]
    EPAD = E
    npe = EPAD - E
    if npe:
        assert NPAD > N
        ar = jnp.arange(npe, dtype=jnp.int32)
        src = jnp.concatenate([src, ar % N])
        dst = jnp.concatenate([dst, N + ar % (NPAD - N)])

    degp = _make_deg_kernel(EPAD, NPAD)(dst)
    scatter = _make_scatter_kernel(EPAD, NPAD, CW)
    src2 = src
    dst2 = dst

    g1 = _tc_encode(x, W1, b1, Wg1, degp)
    s1 = scatter(g1.reshape(N * NCH, CW), src2, dst2)
    g2 = _tc_mid(s1, g1, degp, bg1, Wg2)
    s2 = scatter(g2.reshape(N * NCH, CW), src2, dst2)
    loss, logits = _tc_head(s2, g2, degp, bg2, batch, sact_cids, Wh, bh)
    return loss.reshape(()), logits
